# baseline (speedup 1.0000x reference)
import functools

import jax
import jax.numpy as jnp
from jax import lax
from jax.experimental import pallas as pl
from jax.experimental.pallas import tpu as pltpu
from jax.experimental.pallas import tpu_sc as plsc

N = 50000
E = 800000
NW = 32
NR = 64
RN = 782
NPAD = NR * RN
CAP = 16384
SCH = 2048
NSC = CAP // SCH
CHK = 8000
NCHUNK = E // CHK
PCH = 4000
RNP = 1568
NPOOLPAD = RNP * NW
NC2 = 73
GCHK = 128

_MESH = plsc.VectorSubcoreMesh(core_axis_name="c", subcore_axis_name="s")
_SC_PARAMS = pltpu.CompilerParams(use_tc_tiling_on_sc=False,
                                  needs_layout_passes=False)


def _wid():
    return lax.axis_index("s") * 2 + lax.axis_index("c")


def _iota():
    return lax.iota(jnp.int32, 16)


def _bin_body(dst_hbm, src_hbm, keys_hbm, srcs_hbm, cnt_hbm,
              dbuf, sbuf, kba, kbb, sba, sbb, cb, sem):
    del sem
    w = _wid()
    it = _iota()
    zi = jnp.zeros((16,), jnp.int32)

    def z(i, _):
        plsc.store_scatter(kba, [i * 16 + it], zi)
        plsc.store_scatter(kbb, [i * 16 + it], zi)
        plsc.store_scatter(sba, [i * 16 + it], zi)
        plsc.store_scatter(sbb, [i * 16 + it], zi)
        return 0

    lax.fori_loop(0, CAP // 16, z, 0)

    lo_a = w * RN
    lo_b = (w + 32) * RN

    def chunk(c, offs):
        off_a, off_b = offs
        pltpu.sync_copy(dst_hbm.at[pl.ds(c * CHK, CHK)], dbuf)
        pltpu.sync_copy(src_hbm.at[pl.ds(c * CHK, CHK)], sbuf)

        def vec(i, offs):
            off_a, off_b = offs
            d = plsc.load_gather(dbuf, [i * 16 + it])
            s = plsc.load_gather(sbuf, [i * 16 + it])
            eid = c * CHK + i * 16 + it
            dla = d - lo_a
            ma = (dla >= 0) & (dla < RN)
            pos_a = off_a + plsc.cumsum(ma.astype(jnp.int32)) - 1
            plsc.store_scatter(kba, [pos_a], dla * (1 << 20) + eid, mask=ma)
            plsc.store_scatter(sba, [pos_a], s, mask=ma)
            off_a = off_a + plsc.all_reduce_population_count(ma)
            dlb = d - lo_b
            mb = (dlb >= 0) & (dlb < RN)
            pos_b = off_b + plsc.cumsum(mb.astype(jnp.int32)) - 1
            plsc.store_scatter(kbb, [pos_b], dlb * (1 << 20) + eid, mask=mb)
            plsc.store_scatter(sbb, [pos_b], s, mask=mb)
            off_b = off_b + plsc.all_reduce_population_count(mb)
            return (off_a, off_b)

        return lax.fori_loop(0, CHK // 16, vec, offs)

    off_a, off_b = lax.fori_loop(0, NCHUNK, chunk, (zi, zi))
    cb[...] = off_a
    pltpu.sync_copy(cb, cnt_hbm.at[w])
    cb[...] = off_b
    pltpu.sync_copy(cb, cnt_hbm.at[w + 32])
    pltpu.sync_copy(kba, keys_hbm.at[w])
    pltpu.sync_copy(kbb, keys_hbm.at[w + 32])
    pltpu.sync_copy(sba, srcs_hbm.at[w])
    pltpu.sync_copy(sbb, srcs_hbm.at[w + 32])


def _bin_edges(dst, src):
    f = pl.kernel(
        _bin_body,
        out_type=[
            jax.ShapeDtypeStruct((NR, CAP), jnp.int32),
            jax.ShapeDtypeStruct((NR, CAP), jnp.int32),
            jax.ShapeDtypeStruct((NR, 16), jnp.int32),
        ],
        mesh=_MESH,
        compiler_params=_SC_PARAMS,
        scratch_types=[
            pltpu.VMEM((CHK,), jnp.int32),
            pltpu.VMEM((CHK,), jnp.int32),
            pltpu.VMEM((CAP,), jnp.int32),
            pltpu.VMEM((CAP,), jnp.int32),
            pltpu.VMEM((CAP,), jnp.int32),
            pltpu.VMEM((CAP,), jnp.int32),
            pltpu.VMEM((16,), jnp.int32),
            pltpu.SemaphoreType.DMA,
        ],
    )
    return f(dst, src)


def _convsc_body(cin, x_hbm, w_hbm, keys_hbm, srcs_hbm, cnt_hbm, s_hbm,
                 kchk, schk, ibuf, xbuf, wbuf, acc, cb, semx, semw, semk):
    lg = {2: 1, 8: 3, 16: 4}[cin]
    accw = 8 * cin
    nb = accw // 16
    w = _wid()
    it = _iota()
    zf = jnp.zeros((16,), jnp.float32)
    xcol = it & (cin - 1)
    bsel = jnp.right_shift(it, lg)

    for rr in range(2):
        r = w + rr * 32

        def z(i, _):
            plsc.store_scatter(acc, [i * 16 + it], zf)
            return 0

        lax.fori_loop(0, RN * accw // 16, z, 0)
        pltpu.sync_copy(cnt_hbm.at[r], cb)
        cnt = cb[...][0]

        def superchunk(sc, _):
            @pl.when(sc * SCH < cnt)
            def _():
                ck = pltpu.async_copy(
                    keys_hbm.at[r].at[pl.ds(sc * SCH, SCH)], kchk, semk)
                cs = pltpu.async_copy(
                    srcs_hbm.at[r].at[pl.ds(sc * SCH, SCH)], schk, semw)
                ck.wait()
                cs.wait()

                def chunk(t, _):
                    base = sc * SCH + t * GCHK

                    @pl.when(base < cnt)
                    def _():
                        def kv(g, _):
                            k16 = plsc.load_gather(
                                kchk, [t * GCHK + g * 16 + it])
                            plsc.store_scatter(ibuf, [g * 16 + it],
                                               k16 & 0xFFFFF)
                            return 0

                        lax.fori_loop(0, GCHK // 16, kv, 0)
                        cx = pltpu.async_copy(
                            x_hbm.at[schk.at[pl.ds(t * GCHK, GCHK)]],
                            xbuf, semx)
                        cw = pltpu.async_copy(w_hbm.at[ibuf], wbuf, semw)
                        cx.wait()
                        cw.wait()

                        def grp_full(g, _):
                            k16 = plsc.load_gather(
                                kchk, [t * GCHK + g * 16 + it])
                            dloc = jnp.right_shift(k16, 20)
                            for j2 in range(16):
                                dsp = dloc[j2]
                                row = jnp.broadcast_to(g * 16 + j2, (16,))
                                xv = plsc.load_gather(xbuf, [row, xcol])
                                for q in range(nb):
                                    wv = plsc.load_gather(
                                        wbuf, [row, q * (16 // cin) + bsel])
                                    plsc.addupdate_scatter(
                                        acc, [dsp * accw + q * 16 + it],
                                        wv * xv)
                            return 0

                        def grp_mask(g, _):
                            k16 = plsc.load_gather(
                                kchk, [t * GCHK + g * 16 + it])
                            dloc = jnp.right_shift(k16, 20)
                            for j2 in range(16):
                                vm = jnp.broadcast_to(
                                    base + g * 16 + j2 < cnt, (16,))
                                dsp = dloc[j2]
                                row = jnp.broadcast_to(g * 16 + j2, (16,))
                                xv = plsc.load_gather(xbuf, [row, xcol])
                                for q in range(nb):
                                    wv = plsc.load_gather(
                                        wbuf, [row, q * (16 // cin) + bsel])
                                    plsc.addupdate_scatter(
                                        acc, [dsp * accw + q * 16 + it],
                                        wv * xv, mask=vm)
                            return 0

                        @pl.when(base + GCHK <= cnt)
                        def _():
                            lax.fori_loop(0, GCHK // 16, grp_full, 0)

                        @pl.when(base + GCHK > cnt)
                        def _():
                            lax.fori_loop(0, GCHK // 16, grp_mask, 0)

                    return 0

                lax.fori_loop(0, SCH // GCHK, chunk, 0)

            return 0

        lax.fori_loop(0, NSC, superchunk, 0)
        pltpu.sync_copy(acc.at[pl.ds(0, RN * accw)],
                        s_hbm.at[pl.ds(r * RN * accw, RN * accw)])


def _convsc(x16, wgt16, keys, srcs, cnts, cin):
    accw = 8 * cin
    f = pl.kernel(
        functools.partial(_convsc_body, cin),
        out_type=[jax.ShapeDtypeStruct((NPAD * accw,), jnp.float32)],
        mesh=_MESH,
        compiler_params=_SC_PARAMS,
        scratch_types=[
            pltpu.VMEM((SCH,), jnp.int32),
            pltpu.VMEM((SCH,), jnp.int32),
            pltpu.VMEM((GCHK,), jnp.int32),
            pltpu.VMEM((GCHK, 16), jnp.float32),
            pltpu.VMEM((GCHK, 16), jnp.float32),
            pltpu.VMEM((RN * accw,), jnp.float32),
            pltpu.VMEM((16,), jnp.int32),
            pltpu.SemaphoreType.DMA,
            pltpu.SemaphoreType.DMA,
            pltpu.SemaphoreType.DMA,
        ],
    )
    return f(x16, wgt16, keys, srcs, cnts)[0].reshape(NPAD, accw)


EBLK = 4000


def _prep_body(ea_ref, w_ref):
    u0 = ea_ref[:, 0:1]
    u1 = ea_ref[:, 1:2]
    u2 = ea_ref[:, 2:3]
    lane = lax.broadcasted_iota(jnp.int32, (EBLK, 16), 1)
    out = jnp.zeros((EBLK, 16), jnp.float32)
    for b in range(8):
        wb = ((u0 if b & 1 else 1.0 - u0)
              * (u1 if (b >> 1) & 1 else 1.0 - u1)
              * (u2 if (b >> 2) & 1 else 1.0 - u2))
        out = jnp.where(lane == b, wb, out)
    w_ref[...] = out


def _prep(edge_attr):
    return pl.pallas_call(
        _prep_body,
        grid=(E // EBLK,),
        in_specs=[pl.BlockSpec((EBLK, 3), lambda i: (i, 0))],
        out_specs=pl.BlockSpec((EBLK, 16), lambda i: (i, 0)),
        out_shape=jax.ShapeDtypeStruct((E, 16), jnp.float32),
    )(edge_attr)


CLB = NPOOLPAD // 8


def _cl_body(p_ref, cl_ref):
    xi = (p_ref[:, 0:1] * (1.0 / 16.0)).astype(jnp.int32)
    yi = (p_ref[:, 1:2] * (1.0 / 12.0)).astype(jnp.int32)
    cl_ref[...] = xi * 9 + yi


def _cl_kernel(posp):
    return pl.pallas_call(
        _cl_body,
        grid=(8,),
        in_specs=[pl.BlockSpec((CLB, 4), lambda i: (i, 0))],
        out_specs=pl.BlockSpec((CLB, 1), lambda i: (i, 0)),
        out_shape=jax.ShapeDtypeStruct((NPOOLPAD, 1), jnp.int32),
    )(posp)


PB = 6256
PSTEPS = NPAD // PB


def _elu(x):
    return jnp.where(x > 0, x, jnp.exp(x) - 1.0)


def _statsA_body(cout, mkdeg, s_ref, deg_ref, wc_ref, e_ref, sum_ref, sq_ref,
                 dego_ref=None):
    s = jnp.dot(s_ref[...], wc_ref[...], preferred_element_type=jnp.float32)
    if mkdeg:
        deg = jnp.clip(s[:, 8:9], 1.0, None)
        dego_ref[...] = deg
    else:
        deg = deg_ref[...]
    e = _elu(s[:, :cout] / deg)
    e_ref[...] = e

    @pl.when(pl.program_id(0) == 0)
    def _():
        sum_ref[...] = jnp.zeros_like(sum_ref)
        sq_ref[...] = jnp.zeros_like(sq_ref)

    sum_ref[...] += jnp.sum(e, axis=0, keepdims=True)
    sq_ref[...] += jnp.sum(e * e, axis=0, keepdims=True)


def _normB_body(cout, add_res, e_ref, sum_ref, sq_ref, g_ref, b_ref, *rest):
    if add_res:
        res_ref, h_ref = rest
    else:
        (h_ref,) = rest
    mu = sum_ref[...] * (1.0 / N)
    var = sq_ref[...] * (1.0 / N) - mu * mu
    h = ((e_ref[...] - mu) * jax.lax.rsqrt(var + 1e-5)
         * g_ref[...] + b_ref[...])
    if add_res:
        h = h + res_ref[...]
    h_ref[...] = h


def _post(s128, wcat, deg, gamma, beta, cout, res=None, mkdeg=False,
          swidth=128):
    wpad = max(cout, 16)
    ins = [s128, deg if deg is not None else s128[:, :1], wcat]
    outs = pl.pallas_call(
        functools.partial(_statsA_body, cout, mkdeg),
        grid=(PSTEPS,),
        in_specs=[
            pl.BlockSpec((PB, swidth), lambda i: (i, 0)),
            pl.BlockSpec((PB, 1), lambda i: (i, 0)),
            pl.BlockSpec((swidth, wpad), lambda i: (0, 0)),
        ],
        out_specs=[
            pl.BlockSpec((PB, cout), lambda i: (i, 0)),
            pl.BlockSpec((1, cout), lambda i: (0, 0)),
            pl.BlockSpec((1, cout), lambda i: (0, 0)),
        ] + ([pl.BlockSpec((PB, 1), lambda i: (i, 0))] if mkdeg else []),
        out_shape=[
            jax.ShapeDtypeStruct((NPAD, cout), jnp.float32),
            jax.ShapeDtypeStruct((1, cout), jnp.float32),
            jax.ShapeDtypeStruct((1, cout), jnp.float32),
        ] + ([jax.ShapeDtypeStruct((NPAD, 1), jnp.float32)] if mkdeg else []),
    )(*ins)
    if mkdeg:
        e, se, sq, dego = outs
    else:
        e, se, sq = outs
        dego = None
    args = [e, se, sq, gamma.reshape(1, cout), beta.reshape(1, cout)]
    in_specs = [
        pl.BlockSpec((PB, cout), lambda i: (i, 0)),
        pl.BlockSpec((1, cout), lambda i: (0, 0)),
        pl.BlockSpec((1, cout), lambda i: (0, 0)),
        pl.BlockSpec((1, cout), lambda i: (0, 0)),
        pl.BlockSpec((1, cout), lambda i: (0, 0)),
    ]
    if res is not None:
        args.append(res)
        in_specs.append(pl.BlockSpec((PB, cout), lambda i: (i, 0)))
    h = pl.pallas_call(
        functools.partial(_normB_body, cout, res is not None),
        grid=(PSTEPS,),
        in_specs=in_specs,
        out_specs=pl.BlockSpec((PB, cout), lambda i: (i, 0)),
        out_shape=jax.ShapeDtypeStruct((NPAD, cout), jnp.float32),
    )(*args)
    return h, dego


def _poolpres_body(h_hbm, pos_hbm, cl_hbm, src_hbm, dst_hbm,
                   hp_hbm, cnt_hbm, ps_hbm, pres_hbm,
                   hbuf, pbuf, clbuf, hpm, cntb, psb, clfull, sbuf, dbuf, pb,
                   sem):
    del sem
    w = _wid()
    it = _iota()
    base = w * RNP
    pltpu.sync_copy(h_hbm.at[pl.ds(base * 32, RNP * 32)], hbuf)
    pltpu.sync_copy(pos_hbm.at[pl.ds(base * 4, RNP * 4)],
                    pbuf.at[pl.ds(0, RNP * 4)])
    pltpu.sync_copy(cl_hbm.at[pl.ds(base, RNP)], clbuf)

    def z(i, _):
        plsc.store_scatter(hpm, [i * 16 + it],
                           jnp.full((16,), -1e30, jnp.float32))
        return 0

    lax.fori_loop(0, NC2 * 32 // 16, z, 0)

    def z2(i, _):
        plsc.store_scatter(psb, [i * 16 + it], jnp.zeros((16,), jnp.float32))
        return 0

    lax.fori_loop(0, NC2 * 16 // 16, z2, 0)

    def z3(i, _):
        plsc.store_scatter(cntb, [i * 16 + it], jnp.zeros((16,), jnp.float32))
        return 0

    lax.fori_loop(0, 80 // 16, z3, 0)

    def vec(i, _):
        nloc = i * 16 + it
        cl0 = plsc.load_gather(clbuf, [nloc])
        clv = jnp.where(base + nloc < N, cl0, 72)

        def node(j2):
            c = clv[j2]
            nl = i * 16 + j2
            for v in range(2):
                idx = c * 32 + v * 16 + it
                old = plsc.load_gather(hpm, [idx])
                hrow = plsc.load_gather(hbuf, [nl * 32 + v * 16 + it])
                plsc.store_scatter(hpm, [idx], jnp.maximum(old, hrow))
            pidx = c * 16 + it
            pold = plsc.load_gather(psb, [pidx])
            prow = plsc.load_gather(pbuf, [nl * 4 + it], mask=it < 4)
            plsc.store_scatter(psb, [pidx], pold + prow, mask=it < 3)
            cidx = jnp.broadcast_to(c, (16,))
            cold = plsc.load_gather(cntb, [cidx], mask=it < 1)
            plsc.store_scatter(cntb, [cidx], cold + 1.0, mask=it < 1)

        for j2 in range(16):
            node(j2)
        return 0

    lax.fori_loop(0, RNP // 16, vec, 0)

    pltpu.sync_copy(hpm, hp_hbm.at[w])
    pltpu.sync_copy(cntb, cnt_hbm.at[w])
    pltpu.sync_copy(psb, ps_hbm.at[w])

    pltpu.sync_copy(cl_hbm, clfull)

    def zp(i, _):
        plsc.store_scatter(pb, [i * 16 + it], jnp.zeros((16,), jnp.float32))
        return 0

    lax.fori_loop(0, 5184 // 16, zp, 0)

    def pchunk(k, _):
        cid = w + 32 * k

        @pl.when(cid < E // PCH)
        def _():
            pltpu.sync_copy(src_hbm.at[pl.ds(cid * PCH, PCH)], sbuf)
            pltpu.sync_copy(dst_hbm.at[pl.ds(cid * PCH, PCH)], dbuf)

            def pvec(i, _):
                s = plsc.load_gather(sbuf, [i * 16 + it])
                d = plsc.load_gather(dbuf, [i * 16 + it])
                cs = plsc.load_gather(clfull, [s])
                cd = plsc.load_gather(clfull, [d])
                m = cs != cd
                plsc.store_scatter(pb, [cd * 72 + cs],
                                   jnp.ones((16,), jnp.float32), mask=m)
                return 0

            lax.fori_loop(0, PCH // 16, pvec, 0)

        return 0

    lax.fori_loop(0, (E // PCH + 31) // 32, pchunk, 0)
    pltpu.sync_copy(pb, pres_hbm.at[w])


def _poolpres(h5p, posp, cl, src, dst):
    f = pl.kernel(
        _poolpres_body,
        out_type=[
            jax.ShapeDtypeStruct((NW, NC2 * 32), jnp.float32),
            jax.ShapeDtypeStruct((NW, 80), jnp.float32),
            jax.ShapeDtypeStruct((NW, NC2 * 16), jnp.float32),
            jax.ShapeDtypeStruct((NW, 5184), jnp.float32),
        ],
        mesh=_MESH,
        compiler_params=_SC_PARAMS,
        scratch_types=[
            pltpu.VMEM((RNP * 32,), jnp.float32),
            pltpu.VMEM((RNP * 4 + 16,), jnp.float32),
            pltpu.VMEM((RNP,), jnp.int32),
            pltpu.VMEM((NC2 * 32,), jnp.float32),
            pltpu.VMEM((80,), jnp.float32),
            pltpu.VMEM((NC2 * 16,), jnp.float32),
            pltpu.VMEM((NPOOLPAD,), jnp.int32),
            pltpu.VMEM((PCH,), jnp.int32),
            pltpu.VMEM((PCH,), jnp.int32),
            pltpu.VMEM((5184,), jnp.float32),
            pltpu.SemaphoreType.DMA,
        ],
    )
    return f(h5p, posp, cl, src, dst)


def _stage2_body(hp_ref, cnt_ref, ps_ref, pres_ref, w6_ref, w7_ref,
                 g6_ref, b6_ref, g7_ref, b7_ref, fcw_ref, out_ref):
    hp = hp_ref[0:NC2]
    cnt = cnt_ref[0:1]
    ps = ps_ref[0:NC2]
    presT = pres_ref[0:72]
    for k in range(1, NW):
        hp = jnp.maximum(hp, hp_ref[k * NC2:(k + 1) * NC2])
        cnt = cnt + cnt_ref[k:k + 1]
        ps = ps + ps_ref[k * NC2:(k + 1) * NC2]
        presT = jnp.maximum(presT, pres_ref[k * 72:(k + 1) * 72])
    hp = hp[:72]
    hp = jnp.where(hp < -1e29, 0.0, hp)
    cnt = jnp.clip(cnt[0, :72], 1.0, None)
    pos_p = ps[:72, :3] / cnt[:, None]

    px, py, pz = pos_p[:, 0], pos_p[:, 1], pos_p[:, 2]
    cx = presT * (px[:, None] - px[None, :])
    cy = presT * (py[:, None] - py[None, :])
    cz = presT * (pz[:, None] - pz[None, :])
    mx = jnp.maximum(jnp.max(jnp.abs(cx)),
                     jnp.maximum(jnp.max(jnp.abs(cy)), jnp.max(jnp.abs(cz))))
    den = 1.0 / (2.0 * mx + 1e-12)
    u0 = cx * den + 0.5
    u1 = cy * den + 0.5
    u2 = cz * den + 0.5
    deg2 = jnp.clip(jnp.sum(presT, axis=1), 1.0, None)[:, None]

    def bn(x, gamma, beta):
        mu = jnp.mean(x, axis=0)
        var = jnp.mean((x - mu) ** 2, axis=0)
        return (x - mu) * jax.lax.rsqrt(var + 1e-5) * gamma + beta

    def conv2(hin, w_ref):
        acc = jnp.zeros((72, 32), jnp.float32)
        for b in range(8):
            ab = ((u0 if b & 1 else 1.0 - u0)
                  * (u1 if (b >> 1) & 1 else 1.0 - u1)
                  * (u2 if (b >> 2) & 1 else 1.0 - u2)) * presT
            hb = jnp.dot(hin, w_ref[b], preferred_element_type=jnp.float32)
            acc = acc + jnp.dot(ab, hb, preferred_element_type=jnp.float32)
        acc = acc / deg2
        return jnp.where(acc > 0, acc, jnp.exp(acc) - 1.0)

    h2 = bn(conv2(hp, w6_ref), g6_ref[0], b6_ref[0])
    h2 = bn(conv2(h2, w7_ref), g7_ref[0], b7_ref[0]) + hp

    gx = jnp.clip((pos_p[:, 0] * (1.0 / 30.0)).astype(jnp.int32), 0, 3)
    gy = jnp.clip((pos_p[:, 1] * (1.0 / 25.0)).astype(jnp.int32), 0, 3)
    c7 = gx * 4 + gy
    r = jnp.zeros((1, 2), jnp.float32)
    for k in range(16):
        mk = (c7 == k)[:, None]
        vk = jnp.max(jnp.where(mk, h2, -1e30), axis=0)
        vk = jnp.where(vk < -1e29, 0.0, vk)
        r = r + jnp.sum(fcw_ref[:, k, :] * vk[None, :],
                        axis=1).reshape(1, 2)
    out_ref[...] = r


def _stage2(hp_part, cnt_part, ps_part, pres_part, W6, W7,
            g6, b6, g7, b7, fc_w):
    return pl.pallas_call(
        _stage2_body,
        out_shape=jax.ShapeDtypeStruct((1, 2), jnp.float32),
    )(hp_part.reshape(NW * NC2, 32), cnt_part, ps_part.reshape(NW * NC2, 16),
      pres_part.reshape(NW * 72, 72), W6, W7,
      g6.reshape(1, 32), b6.reshape(1, 32),
      g7.reshape(1, 32), b7.reshape(1, 32),
      fc_w.reshape(2, 16, 32))


def _wcat(W, cin, cpad, deg_col=False):
    c_in, c_out = W.shape[1], W.shape[2]
    w = jnp.zeros((8, cin, cpad), jnp.float32)
    w = w.at[:, :c_in, :c_out].set(W)
    if deg_col:
        w = w.at[:, 1, 8].set(1.0)
    return w.reshape(8 * cin, cpad)


def kernel(x, edge_index, edge_attr, pos, batch, W1, W2, W3, W4, W5, W6, W7,
           g1, g2, g3, g4, g5, g6, g7, b1, b2, b3, b4, b5, b6, b7, fc_w):
    src = edge_index[0].astype(jnp.int32)
    dst = edge_index[1].astype(jnp.int32)

    keys, srcs, cnts = _bin_edges(dst, src)
    wgt16 = _prep(edge_attr)

    def conv(xin, cin):
        return _convsc(xin, wgt16, keys, srcs, cnts, cin)

    x16 = jnp.zeros((NPAD, 16), jnp.float32)
    x16 = x16.at[:N, 0].set(x[:, 0]).at[:N, 1].set(1.0)
    s1 = conv(x16, 2)
    h, deg = _post(s1, _wcat(W1, 2, 16, deg_col=True), None, g1, b1, 8,
                   mkdeg=True, swidth=16)
    h16 = jnp.zeros((NPAD, 16), jnp.float32).at[:, :8].set(h)

    s2 = conv(h16, 8)
    h, _ = _post(s2, _wcat(W2, 8, 16), deg, g2, b2, 16, swidth=64)
    x_sc = h
    s3 = conv(h, 16)
    h, _ = _post(s3, _wcat(W3, 16, 16), deg, g3, b3, 16)
    s4 = conv(h, 16)
    h, _ = _post(s4, _wcat(W4, 16, 16), deg, g4, b4, 16, res=x_sc)
    s5 = conv(h, 16)
    h5, _ = _post(s5, _wcat(W5, 16, 32), deg, g5, b5, 32)

    h5p = jnp.zeros((NPOOLPAD, 32), jnp.float32).at[:NPAD].set(h5)
    posp = jnp.zeros((NPOOLPAD, 4), jnp.float32).at[:N, :3].set(pos)
    cl = _cl_kernel(posp).reshape(-1)
    hp_part, cnt_part, ps_part, pres_part = _poolpres(
        h5p.reshape(-1), posp.reshape(-1), cl, src, dst)

    return _stage2(hp_part, cnt_part, ps_part, pres_part, W6, W7,
                   g6, b6, g7, b7, fc_w)


# speedup vs baseline: 1.4395x; 1.4395x over previous
import functools

import jax
import jax.numpy as jnp
from jax import lax
from jax.experimental import pallas as pl
from jax.experimental.pallas import tpu as pltpu
from jax.experimental.pallas import tpu_sc as plsc

N = 50000
E = 800000
NW = 32
NR = 64
RN = 782
NPAD = NR * RN
CAP = 16384
SCH = 2048
NSC = CAP // SCH
CHK = 8000
NCHUNK = E // CHK
PCH = 4000
RNP = 1568
NPOOLPAD = RNP * NW
NC2 = 73
GCHK = 128

_MESH = plsc.VectorSubcoreMesh(core_axis_name="c", subcore_axis_name="s")
_SC_PARAMS = pltpu.CompilerParams(use_tc_tiling_on_sc=False,
                                  needs_layout_passes=False)


def _wid():
    return lax.axis_index("s") * 2 + lax.axis_index("c")


def _iota():
    return lax.iota(jnp.int32, 16)


def _bin_body(dst_hbm, src_hbm, keys_hbm, srcs_hbm, cnt_hbm,
              dbuf, sbuf, kba, kbb, sba, sbb, cb, sem):
    del sem
    w = _wid()
    it = _iota()
    zi = jnp.zeros((16,), jnp.int32)

    def z(i, _):
        plsc.store_scatter(kba, [i * 16 + it], zi)
        plsc.store_scatter(kbb, [i * 16 + it], zi)
        plsc.store_scatter(sba, [i * 16 + it], zi)
        plsc.store_scatter(sbb, [i * 16 + it], zi)
        return 0

    lax.fori_loop(0, CAP // 16, z, 0)

    lo_a = w * RN
    lo_b = (w + 32) * RN

    def chunk(c, offs):
        off_a, off_b = offs
        pltpu.sync_copy(dst_hbm.at[pl.ds(c * CHK, CHK)], dbuf)
        pltpu.sync_copy(src_hbm.at[pl.ds(c * CHK, CHK)], sbuf)

        def vec(i, offs):
            off_a, off_b = offs
            d = plsc.load_gather(dbuf, [i * 16 + it])
            s = plsc.load_gather(sbuf, [i * 16 + it])
            eid = c * CHK + i * 16 + it
            dla = d - lo_a
            ma = (dla >= 0) & (dla < RN)
            pos_a = off_a + plsc.cumsum(ma.astype(jnp.int32)) - 1
            plsc.store_scatter(kba, [pos_a], dla * (1 << 20) + eid, mask=ma)
            plsc.store_scatter(sba, [pos_a], s, mask=ma)
            off_a = off_a + plsc.all_reduce_population_count(ma)
            dlb = d - lo_b
            mb = (dlb >= 0) & (dlb < RN)
            pos_b = off_b + plsc.cumsum(mb.astype(jnp.int32)) - 1
            plsc.store_scatter(kbb, [pos_b], dlb * (1 << 20) + eid, mask=mb)
            plsc.store_scatter(sbb, [pos_b], s, mask=mb)
            off_b = off_b + plsc.all_reduce_population_count(mb)
            return (off_a, off_b)

        return lax.fori_loop(0, CHK // 16, vec, offs)

    off_a, off_b = lax.fori_loop(0, NCHUNK, chunk, (zi, zi))
    cb[...] = off_a
    pltpu.sync_copy(cb, cnt_hbm.at[w])
    cb[...] = off_b
    pltpu.sync_copy(cb, cnt_hbm.at[w + 32])
    pltpu.sync_copy(kba, keys_hbm.at[w])
    pltpu.sync_copy(kbb, keys_hbm.at[w + 32])
    pltpu.sync_copy(sba, srcs_hbm.at[w])
    pltpu.sync_copy(sbb, srcs_hbm.at[w + 32])


def _bin_edges(dst, src):
    f = pl.kernel(
        _bin_body,
        out_type=[
            jax.ShapeDtypeStruct((NR, CAP), jnp.int32),
            jax.ShapeDtypeStruct((NR, CAP), jnp.int32),
            jax.ShapeDtypeStruct((NR, 16), jnp.int32),
        ],
        mesh=_MESH,
        compiler_params=_SC_PARAMS,
        scratch_types=[
            pltpu.VMEM((CHK,), jnp.int32),
            pltpu.VMEM((CHK,), jnp.int32),
            pltpu.VMEM((CAP,), jnp.int32),
            pltpu.VMEM((CAP,), jnp.int32),
            pltpu.VMEM((CAP,), jnp.int32),
            pltpu.VMEM((CAP,), jnp.int32),
            pltpu.VMEM((16,), jnp.int32),
            pltpu.SemaphoreType.DMA,
        ],
    )
    return f(dst, src)


def _convsc_body(x_hbm, w_hbm, keys_hbm, srcs_hbm, cnt_hbm, s_hbm,
                 kchk, schk, ibuf, xbuf, wbuf, acc, cb, semx, semw, semk):
    w = _wid()
    it = _iota()
    zf = jnp.zeros((16,), jnp.float32)

    for rr in range(2):
        r = w + rr * 32

        def z(i, _):
            plsc.store_scatter(acc, [i * 16 + it], zf)
            return 0

        lax.fori_loop(0, RN * 128 // 16, z, 0)
        pltpu.sync_copy(cnt_hbm.at[r], cb)
        cnt = cb[...][0]

        def superchunk(sc, _):
            @pl.when(sc * SCH < cnt)
            def _():
                ck = pltpu.async_copy(
                    keys_hbm.at[r].at[pl.ds(sc * SCH, SCH)], kchk, semk)
                cs = pltpu.async_copy(
                    srcs_hbm.at[r].at[pl.ds(sc * SCH, SCH)], schk, semw)
                ck.wait()
                cs.wait()

                def chunk(t, _):
                    base = sc * SCH + t * GCHK

                    @pl.when(base < cnt)
                    def _():
                        def kv(g, _):
                            k16 = plsc.load_gather(
                                kchk, [t * GCHK + g * 16 + it])
                            plsc.store_scatter(ibuf, [g * 16 + it],
                                               k16 & 0xFFFFF)
                            return 0

                        lax.fori_loop(0, GCHK // 16, kv, 0)
                        cx = pltpu.async_copy(
                            x_hbm.at[schk.at[pl.ds(t * GCHK, GCHK)]],
                            xbuf, semx)
                        cw = pltpu.async_copy(w_hbm.at[ibuf], wbuf, semw)
                        cx.wait()
                        cw.wait()

                        def grp_full(g, _):
                            k16 = plsc.load_gather(
                                kchk, [t * GCHK + g * 16 + it])
                            dloc = jnp.right_shift(k16, 20)
                            for j2 in range(16):
                                dsp = dloc[j2]
                                row = jnp.broadcast_to(g * 16 + j2, (16,))
                                xrow = plsc.load_gather(xbuf, [row, it])
                                w16 = plsc.load_gather(wbuf, [row, it])
                                for b in range(8):
                                    plsc.addupdate_scatter(
                                        acc, [dsp * 128 + b * 16 + it],
                                        w16[b] * xrow)
                            return 0

                        def grp_mask(g, _):
                            k16 = plsc.load_gather(
                                kchk, [t * GCHK + g * 16 + it])
                            dloc = jnp.right_shift(k16, 20)
                            for j2 in range(16):
                                vm = jnp.broadcast_to(
                                    base + g * 16 + j2 < cnt, (16,))
                                dsp = dloc[j2]
                                row = jnp.broadcast_to(g * 16 + j2, (16,))
                                xrow = plsc.load_gather(xbuf, [row, it])
                                w16 = plsc.load_gather(wbuf, [row, it])
                                for b in range(8):
                                    plsc.addupdate_scatter(
                                        acc, [dsp * 128 + b * 16 + it],
                                        w16[b] * xrow, mask=vm)
                            return 0

                        @pl.when(base + GCHK <= cnt)
                        def _():
                            lax.fori_loop(0, GCHK // 16, grp_full, 0)

                        @pl.when(base + GCHK > cnt)
                        def _():
                            lax.fori_loop(0, GCHK // 16, grp_mask, 0)

                    return 0

                lax.fori_loop(0, SCH // GCHK, chunk, 0)

            return 0

        lax.fori_loop(0, NSC, superchunk, 0)
        pltpu.sync_copy(acc, s_hbm.at[pl.ds(r * RN * 128, RN * 128)])


def _convsc(x16, wgt16, keys, srcs, cnts):
    f = pl.kernel(
        _convsc_body,
        out_type=[jax.ShapeDtypeStruct((NPAD * 128,), jnp.float32)],
        mesh=_MESH,
        compiler_params=_SC_PARAMS,
        scratch_types=[
            pltpu.VMEM((SCH,), jnp.int32),
            pltpu.VMEM((SCH,), jnp.int32),
            pltpu.VMEM((GCHK,), jnp.int32),
            pltpu.VMEM((GCHK, 16), jnp.float32),
            pltpu.VMEM((GCHK, 16), jnp.float32),
            pltpu.VMEM((RN * 128,), jnp.float32),
            pltpu.VMEM((16,), jnp.int32),
            pltpu.SemaphoreType.DMA,
            pltpu.SemaphoreType.DMA,
            pltpu.SemaphoreType.DMA,
        ],
    )
    return f(x16, wgt16, keys, srcs, cnts)[0].reshape(NPAD, 128)


EBLK = 4000


def _prep_body(ea_ref, w_ref):
    u0 = ea_ref[:, 0:1]
    u1 = ea_ref[:, 1:2]
    u2 = ea_ref[:, 2:3]
    lane = lax.broadcasted_iota(jnp.int32, (EBLK, 16), 1)
    out = jnp.zeros((EBLK, 16), jnp.float32)
    for b in range(8):
        wb = ((u0 if b & 1 else 1.0 - u0)
              * (u1 if (b >> 1) & 1 else 1.0 - u1)
              * (u2 if (b >> 2) & 1 else 1.0 - u2))
        out = jnp.where(lane == b, wb, out)
    w_ref[...] = out


def _prep(edge_attr):
    return pl.pallas_call(
        _prep_body,
        grid=(E // EBLK,),
        in_specs=[pl.BlockSpec((EBLK, 3), lambda i: (i, 0))],
        out_specs=pl.BlockSpec((EBLK, 16), lambda i: (i, 0)),
        out_shape=jax.ShapeDtypeStruct((E, 16), jnp.float32),
    )(edge_attr)


CLB = NPOOLPAD // 8


def _cl_body(p_ref, cl_ref):
    xi = (p_ref[:, 0:1] * (1.0 / 16.0)).astype(jnp.int32)
    yi = (p_ref[:, 1:2] * (1.0 / 12.0)).astype(jnp.int32)
    cl_ref[...] = xi * 9 + yi


def _cl_kernel(posp):
    return pl.pallas_call(
        _cl_body,
        grid=(8,),
        in_specs=[pl.BlockSpec((CLB, 4), lambda i: (i, 0))],
        out_specs=pl.BlockSpec((CLB, 1), lambda i: (i, 0)),
        out_shape=jax.ShapeDtypeStruct((NPOOLPAD, 1), jnp.int32),
    )(posp)


PB = 6256
PSTEPS = NPAD // PB


def _elu(x):
    return jnp.where(x > 0, x, jnp.exp(x) - 1.0)


def _statsA_body(cout, mkdeg, s_ref, deg_ref, wc_ref, e_ref, sum_ref, sq_ref,
                 dego_ref=None):
    s = jnp.dot(s_ref[...], wc_ref[...], preferred_element_type=jnp.float32)
    if mkdeg:
        deg = jnp.clip(s[:, 8:9], 1.0, None)
        dego_ref[...] = deg
    else:
        deg = deg_ref[...]
    e = _elu(s[:, :cout] / deg)
    e_ref[...] = e

    @pl.when(pl.program_id(0) == 0)
    def _():
        sum_ref[...] = jnp.zeros_like(sum_ref)
        sq_ref[...] = jnp.zeros_like(sq_ref)

    sum_ref[...] += jnp.sum(e, axis=0, keepdims=True)
    sq_ref[...] += jnp.sum(e * e, axis=0, keepdims=True)


def _normB_body(cout, add_res, e_ref, sum_ref, sq_ref, g_ref, b_ref, *rest):
    if add_res:
        res_ref, h_ref = rest
    else:
        (h_ref,) = rest
    mu = sum_ref[...] * (1.0 / N)
    var = sq_ref[...] * (1.0 / N) - mu * mu
    h = ((e_ref[...] - mu) * jax.lax.rsqrt(var + 1e-5)
         * g_ref[...] + b_ref[...])
    if add_res:
        h = h + res_ref[...]
    h_ref[...] = h


def _post(s128, wcat, deg, gamma, beta, cout, res=None, mkdeg=False):
    wpad = max(cout, 16)
    ins = [s128, deg if deg is not None else s128[:, :1], wcat]
    outs = pl.pallas_call(
        functools.partial(_statsA_body, cout, mkdeg),
        grid=(PSTEPS,),
        in_specs=[
            pl.BlockSpec((PB, 128), lambda i: (i, 0)),
            pl.BlockSpec((PB, 1), lambda i: (i, 0)),
            pl.BlockSpec((128, wpad), lambda i: (0, 0)),
        ],
        out_specs=[
            pl.BlockSpec((PB, cout), lambda i: (i, 0)),
            pl.BlockSpec((1, cout), lambda i: (0, 0)),
            pl.BlockSpec((1, cout), lambda i: (0, 0)),
        ] + ([pl.BlockSpec((PB, 1), lambda i: (i, 0))] if mkdeg else []),
        out_shape=[
            jax.ShapeDtypeStruct((NPAD, cout), jnp.float32),
            jax.ShapeDtypeStruct((1, cout), jnp.float32),
            jax.ShapeDtypeStruct((1, cout), jnp.float32),
        ] + ([jax.ShapeDtypeStruct((NPAD, 1), jnp.float32)] if mkdeg else []),
    )(*ins)
    if mkdeg:
        e, se, sq, dego = outs
    else:
        e, se, sq = outs
        dego = None
    args = [e, se, sq, gamma.reshape(1, cout), beta.reshape(1, cout)]
    in_specs = [
        pl.BlockSpec((PB, cout), lambda i: (i, 0)),
        pl.BlockSpec((1, cout), lambda i: (0, 0)),
        pl.BlockSpec((1, cout), lambda i: (0, 0)),
        pl.BlockSpec((1, cout), lambda i: (0, 0)),
        pl.BlockSpec((1, cout), lambda i: (0, 0)),
    ]
    if res is not None:
        args.append(res)
        in_specs.append(pl.BlockSpec((PB, cout), lambda i: (i, 0)))
    h = pl.pallas_call(
        functools.partial(_normB_body, cout, res is not None),
        grid=(PSTEPS,),
        in_specs=in_specs,
        out_specs=pl.BlockSpec((PB, cout), lambda i: (i, 0)),
        out_shape=jax.ShapeDtypeStruct((NPAD, cout), jnp.float32),
    )(*args)
    return h, dego


def _poolpres_body(h_hbm, pos_hbm, cl_hbm, src_hbm, dst_hbm,
                   hp_hbm, cnt_hbm, ps_hbm, pres_hbm,
                   hbuf, pbuf, clbuf, hpm, cntb, psb, clfull, sbuf, dbuf, pb,
                   sem):
    del sem
    w = _wid()
    it = _iota()
    base = w * RNP
    pltpu.sync_copy(h_hbm.at[pl.ds(base * 32, RNP * 32)], hbuf)
    pltpu.sync_copy(pos_hbm.at[pl.ds(base * 4, RNP * 4)],
                    pbuf.at[pl.ds(0, RNP * 4)])
    pltpu.sync_copy(cl_hbm.at[pl.ds(base, RNP)], clbuf)

    def z(i, _):
        plsc.store_scatter(hpm, [i * 16 + it],
                           jnp.full((16,), -1e30, jnp.float32))
        return 0

    lax.fori_loop(0, NC2 * 32 // 16, z, 0)

    def z2(i, _):
        plsc.store_scatter(psb, [i * 16 + it], jnp.zeros((16,), jnp.float32))
        return 0

    lax.fori_loop(0, NC2 * 16 // 16, z2, 0)

    def z3(i, _):
        plsc.store_scatter(cntb, [i * 16 + it], jnp.zeros((16,), jnp.float32))
        return 0

    lax.fori_loop(0, 80 // 16, z3, 0)

    def vec(i, _):
        nloc = i * 16 + it
        cl0 = plsc.load_gather(clbuf, [nloc])
        clv = jnp.where(base + nloc < N, cl0, 72)

        def node(j2):
            c = clv[j2]
            nl = i * 16 + j2
            for v in range(2):
                idx = c * 32 + v * 16 + it
                old = plsc.load_gather(hpm, [idx])
                hrow = plsc.load_gather(hbuf, [nl * 32 + v * 16 + it])
                plsc.store_scatter(hpm, [idx], jnp.maximum(old, hrow))
            pidx = c * 16 + it
            pold = plsc.load_gather(psb, [pidx])
            prow = plsc.load_gather(pbuf, [nl * 4 + it], mask=it < 4)
            plsc.store_scatter(psb, [pidx], pold + prow, mask=it < 3)
            cidx = jnp.broadcast_to(c, (16,))
            cold = plsc.load_gather(cntb, [cidx], mask=it < 1)
            plsc.store_scatter(cntb, [cidx], cold + 1.0, mask=it < 1)

        for j2 in range(16):
            node(j2)
        return 0

    lax.fori_loop(0, RNP // 16, vec, 0)

    pltpu.sync_copy(hpm, hp_hbm.at[w])
    pltpu.sync_copy(cntb, cnt_hbm.at[w])
    pltpu.sync_copy(psb, ps_hbm.at[w])

    pltpu.sync_copy(cl_hbm, clfull)

    def zp(i, _):
        plsc.store_scatter(pb, [i * 16 + it], jnp.zeros((16,), jnp.float32))
        return 0

    lax.fori_loop(0, 5184 // 16, zp, 0)

    def pchunk(k, _):
        cid = w + 32 * k

        @pl.when(cid < E // PCH)
        def _():
            pltpu.sync_copy(src_hbm.at[pl.ds(cid * PCH, PCH)], sbuf)
            pltpu.sync_copy(dst_hbm.at[pl.ds(cid * PCH, PCH)], dbuf)

            def pvec(i, _):
                s = plsc.load_gather(sbuf, [i * 16 + it])
                d = plsc.load_gather(dbuf, [i * 16 + it])
                cs = plsc.load_gather(clfull, [s])
                cd = plsc.load_gather(clfull, [d])
                m = cs != cd
                plsc.store_scatter(pb, [cd * 72 + cs],
                                   jnp.ones((16,), jnp.float32), mask=m)
                return 0

            lax.fori_loop(0, PCH // 16, pvec, 0)

        return 0

    lax.fori_loop(0, (E // PCH + 31) // 32, pchunk, 0)
    pltpu.sync_copy(pb, pres_hbm.at[w])


def _poolpres(h5p, posp, cl, src, dst):
    f = pl.kernel(
        _poolpres_body,
        out_type=[
            jax.ShapeDtypeStruct((NW, NC2 * 32), jnp.float32),
            jax.ShapeDtypeStruct((NW, 80), jnp.float32),
            jax.ShapeDtypeStruct((NW, NC2 * 16), jnp.float32),
            jax.ShapeDtypeStruct((NW, 5184), jnp.float32),
        ],
        mesh=_MESH,
        compiler_params=_SC_PARAMS,
        scratch_types=[
            pltpu.VMEM((RNP * 32,), jnp.float32),
            pltpu.VMEM((RNP * 4 + 16,), jnp.float32),
            pltpu.VMEM((RNP,), jnp.int32),
            pltpu.VMEM((NC2 * 32,), jnp.float32),
            pltpu.VMEM((80,), jnp.float32),
            pltpu.VMEM((NC2 * 16,), jnp.float32),
            pltpu.VMEM((NPOOLPAD,), jnp.int32),
            pltpu.VMEM((PCH,), jnp.int32),
            pltpu.VMEM((PCH,), jnp.int32),
            pltpu.VMEM((5184,), jnp.float32),
            pltpu.SemaphoreType.DMA,
        ],
    )
    return f(h5p, posp, cl, src, dst)


def _stage2_body(hp_ref, cnt_ref, ps_ref, pres_ref, w6_ref, w7_ref,
                 g6_ref, b6_ref, g7_ref, b7_ref, fcw_ref, out_ref):
    hp = hp_ref[0:NC2]
    cnt = cnt_ref[0:1]
    ps = ps_ref[0:NC2]
    presT = pres_ref[0:72]
    for k in range(1, NW):
        hp = jnp.maximum(hp, hp_ref[k * NC2:(k + 1) * NC2])
        cnt = cnt + cnt_ref[k:k + 1]
        ps = ps + ps_ref[k * NC2:(k + 1) * NC2]
        presT = jnp.maximum(presT, pres_ref[k * 72:(k + 1) * 72])
    hp = hp[:72]
    hp = jnp.where(hp < -1e29, 0.0, hp)
    cnt = jnp.clip(cnt[0, :72], 1.0, None)
    pos_p = ps[:72, :3] / cnt[:, None]

    px, py, pz = pos_p[:, 0], pos_p[:, 1], pos_p[:, 2]
    cx = presT * (px[:, None] - px[None, :])
    cy = presT * (py[:, None] - py[None, :])
    cz = presT * (pz[:, None] - pz[None, :])
    mx = jnp.maximum(jnp.max(jnp.abs(cx)),
                     jnp.maximum(jnp.max(jnp.abs(cy)), jnp.max(jnp.abs(cz))))
    den = 1.0 / (2.0 * mx + 1e-12)
    u0 = cx * den + 0.5
    u1 = cy * den + 0.5
    u2 = cz * den + 0.5
    deg2 = jnp.clip(jnp.sum(presT, axis=1), 1.0, None)[:, None]

    def bn(x, gamma, beta):
        mu = jnp.mean(x, axis=0)
        var = jnp.mean((x - mu) ** 2, axis=0)
        return (x - mu) * jax.lax.rsqrt(var + 1e-5) * gamma + beta

    def conv2(hin, w_ref):
        acc = jnp.zeros((72, 32), jnp.float32)
        for b in range(8):
            ab = ((u0 if b & 1 else 1.0 - u0)
                  * (u1 if (b >> 1) & 1 else 1.0 - u1)
                  * (u2 if (b >> 2) & 1 else 1.0 - u2)) * presT
            hb = jnp.dot(hin, w_ref[b], preferred_element_type=jnp.float32)
            acc = acc + jnp.dot(ab, hb, preferred_element_type=jnp.float32)
        acc = acc / deg2
        return jnp.where(acc > 0, acc, jnp.exp(acc) - 1.0)

    h2 = bn(conv2(hp, w6_ref), g6_ref[0], b6_ref[0])
    h2 = bn(conv2(h2, w7_ref), g7_ref[0], b7_ref[0]) + hp

    gx = jnp.clip((pos_p[:, 0] * (1.0 / 30.0)).astype(jnp.int32), 0, 3)
    gy = jnp.clip((pos_p[:, 1] * (1.0 / 25.0)).astype(jnp.int32), 0, 3)
    c7 = gx * 4 + gy
    r = jnp.zeros((1, 2), jnp.float32)
    for k in range(16):
        mk = (c7 == k)[:, None]
        vk = jnp.max(jnp.where(mk, h2, -1e30), axis=0)
        vk = jnp.where(vk < -1e29, 0.0, vk)
        r = r + jnp.sum(fcw_ref[:, k, :] * vk[None, :],
                        axis=1).reshape(1, 2)
    out_ref[...] = r


def _stage2(hp_part, cnt_part, ps_part, pres_part, W6, W7,
            g6, b6, g7, b7, fc_w):
    return pl.pallas_call(
        _stage2_body,
        out_shape=jax.ShapeDtypeStruct((1, 2), jnp.float32),
    )(hp_part.reshape(NW * NC2, 32), cnt_part, ps_part.reshape(NW * NC2, 16),
      pres_part.reshape(NW * 72, 72), W6, W7,
      g6.reshape(1, 32), b6.reshape(1, 32),
      g7.reshape(1, 32), b7.reshape(1, 32),
      fc_w.reshape(2, 16, 32))


def _wcat(W, cpad, deg_col=False):
    c_in, c_out = W.shape[1], W.shape[2]
    w = jnp.zeros((8, 16, cpad), jnp.float32)
    w = w.at[:, :c_in, :c_out].set(W)
    if deg_col:
        w = w.at[:, 15, 8].set(1.0)
    return w.reshape(128, cpad)


def kernel(x, edge_index, edge_attr, pos, batch, W1, W2, W3, W4, W5, W6, W7,
           g1, g2, g3, g4, g5, g6, g7, b1, b2, b3, b4, b5, b6, b7, fc_w):
    src = edge_index[0].astype(jnp.int32)
    dst = edge_index[1].astype(jnp.int32)

    keys, srcs, cnts = _bin_edges(dst, src)
    wgt16 = _prep(edge_attr)

    def conv(x16):
        return _convsc(x16, wgt16, keys, srcs, cnts)

    x16 = jnp.concatenate(
        [x, jnp.zeros((N, 14), jnp.float32), jnp.ones((N, 1), jnp.float32)],
        axis=1)
    x16 = jnp.zeros((NPAD, 16), jnp.float32).at[:N].set(x16)
    s1 = conv(x16)
    h, deg = _post(s1, _wcat(W1, 16, deg_col=True), None, g1, b1, 8,
                   mkdeg=True)
    h16 = jnp.zeros((NPAD, 16), jnp.float32).at[:, :8].set(h)

    s2 = conv(h16)
    h, _ = _post(s2, _wcat(W2, 16), deg, g2, b2, 16)
    x_sc = h
    s3 = conv(h)
    h, _ = _post(s3, _wcat(W3, 16), deg, g3, b3, 16)
    s4 = conv(h)
    h, _ = _post(s4, _wcat(W4, 16), deg, g4, b4, 16, res=x_sc)
    s5 = conv(h)
    h5, _ = _post(s5, _wcat(W5, 32), deg, g5, b5, 32)

    h5p = jnp.zeros((NPOOLPAD, 32), jnp.float32).at[:NPAD].set(h5)
    posp = jnp.zeros((NPOOLPAD, 4), jnp.float32).at[:N, :3].set(pos)
    cl = _cl_kernel(posp).reshape(-1)
    hp_part, cnt_part, ps_part, pres_part = _poolpres(
        h5p.reshape(-1), posp.reshape(-1), cl, src, dst)

    return _stage2(hp_part, cnt_part, ps_part, pres_part, W6, W7,
                   g6, b6, g7, b7, fc_w)
